# baseline (device time: 60319 ns/iter reference)
import jax
import jax.numpy as jnp
from jax import lax
from jax.experimental import pallas as pl
from jax.experimental.pallas import tpu as pltpu

N_DEV = 4
B = 64
D = 1024
BG = N_DEV * B
N_PHASE = 6
N_SEM = 3 * N_PHASE


def kernel(x, Win0, Wout0, Win1, Wout1, Win2, Wout2):
    def body(x_ref, win0, wout0, win1, wout1, win2, wout2, out_ref,
             xfull, part, sbuf, rbuf, send_sems, recv_sems):
        my = lax.axis_index("i")
        my_rows = pl.ds(my * B, B)
        half_base = (my // 2) * (2 * B)
        other_base = (1 - my // 2) * (2 * B)

        barrier = pltpu.get_barrier_semaphore()
        for d in (1, 2, 3):
            pl.semaphore_signal(barrier, inc=1, device_id=(my ^ d,),
                                device_id_type=pl.DeviceIdType.MESH)
        pl.semaphore_wait(barrier, 3)

        phase_ctr = [0]

        def make_rdma(ph, d, src, dst):
            import os
            if os.environ.get("SKIP_COMM"):
                class _Noop:
                    def start(self): pass
                    def wait(self): pass
                    def wait_send(self): pass
                    def wait_recv(self): pass
                return _Noop()
            i = 3 * ph + (d - 1)
            return pltpu.make_async_remote_copy(
                src_ref=src, dst_ref=dst,
                send_sem=send_sems.at[i], recv_sem=recv_sems.at[i],
                device_id=(my ^ d,), device_id_type=pl.DeviceIdType.MESH,
            )

        def start_allgather():
            ph = phase_ctr[0]
            phase_ctr[0] += 1
            rdmas = {}
            for d in (1, 2, 3):
                rdmas[d] = make_rdma(ph, d, xfull.at[my_rows, :],
                                     xfull.at[my_rows, :])
                rdmas[d].start()
            return rdmas

        def half_out(win_b, wout_b, base):
            hh = jnp.maximum(
                jnp.dot(xfull[pl.ds(base, 2 * B), :], win_b,
                        preferred_element_type=jnp.float32), 0.0)
            return jnp.dot(hh.astype(jnp.bfloat16), wout_b,
                           preferred_element_type=jnp.float32)

        def stage_and_send(ph, d):
            sbuf[d - 1, :, :] = part[pl.ds((my ^ d) * B, B), :].astype(
                jnp.bfloat16)
            r = make_rdma(ph, d, sbuf.at[d - 1], rbuf.at[d - 1])
            r.start()
            return r

        xfull[my_rows, :] = x_ref[:, :].astype(jnp.bfloat16)
        ag = start_allgather()

        layers = ((win0, wout0), (win1, wout1), (win2, wout2))
        for k, (win, wout) in enumerate(layers):
            win_b = win[:, :].astype(jnp.bfloat16)
            wout_b = wout[:, :].astype(jnp.bfloat16)
            rs = {}
            ph = phase_ctr[0]
            phase_ctr[0] += 1
            ag[1].wait_recv()
            part[pl.ds(half_base, 2 * B), :] = half_out(win_b, wout_b,
                                                        half_base)
            rs[1] = stage_and_send(ph, 1)
            ag[3].wait_recv()
            ag[2].wait_recv()
            part[pl.ds(other_base, 2 * B), :] = half_out(win_b, wout_b,
                                                         other_base)
            rs[3] = stage_and_send(ph, 3)
            rs[2] = stage_and_send(ph, 2)
            for d in (1, 3, 2):
                rs[d].wait_recv()
            red = (part[my_rows, :]
                   + rbuf[0, :, :].astype(jnp.float32)
                   + rbuf[1, :, :].astype(jnp.float32)
                   + rbuf[2, :, :].astype(jnp.float32))
            for d in (1, 2, 3):
                ag[d].wait_send()
                rs[d].wait_send()
            if k < len(layers) - 1:
                xfull[my_rows, :] = red.astype(jnp.bfloat16)
                ag = start_allgather()
            else:
                out_ref[:, :] = red

    return pl.pallas_call(
        body,
        out_shape=jax.ShapeDtypeStruct((B, D), jnp.float32),
        in_specs=[pl.BlockSpec(memory_space=pltpu.VMEM)] * 7,
        out_specs=pl.BlockSpec(memory_space=pltpu.VMEM),
        scratch_shapes=[
            pltpu.VMEM((BG, D), jnp.bfloat16),
            pltpu.VMEM((BG, D), jnp.float32),
            pltpu.VMEM((3, B, D), jnp.bfloat16),
            pltpu.VMEM((3, B, D), jnp.bfloat16),
            pltpu.SemaphoreType.DMA((N_SEM,)),
            pltpu.SemaphoreType.DMA((N_SEM,)),
        ],
        compiler_params=pltpu.CompilerParams(
            collective_id=0,
            vmem_limit_bytes=100 * 1024 * 1024,
        ),
    )(x, Win0, Wout0, Win1, Wout1, Win2, Wout2)


# device time: 59686 ns/iter; 1.0106x vs baseline; 1.0106x over previous
import jax
import jax.numpy as jnp
from jax import lax
from jax.experimental import pallas as pl
from jax.experimental.pallas import tpu as pltpu

N_DEV = 4
B = 64
D = 1024
BG = N_DEV * B
N_PHASE = 6
N_SEM = 3 * N_PHASE


def kernel(x, Win0, Wout0, Win1, Wout1, Win2, Wout2):
    def body(x_ref, win0, wout0, win1, wout1, win2, wout2, out_ref,
             xfull, part, sbuf, rbuf, send_sems, recv_sems):
        my = lax.axis_index("i")
        my_rows = pl.ds(my * B, B)
        half_base = (my // 2) * (2 * B)
        other_base = (1 - my // 2) * (2 * B)

        barrier = pltpu.get_barrier_semaphore()
        for d in (1, 2, 3):
            pl.semaphore_signal(barrier, inc=1, device_id=(my ^ d,),
                                device_id_type=pl.DeviceIdType.MESH)
        pl.semaphore_wait(barrier, 3)

        phase_ctr = [0]

        def make_rdma(ph, d, src, dst):
            import os
            if os.environ.get("SKIP_COMM"):
                class _Noop:
                    def start(self): pass
                    def wait(self): pass
                    def wait_send(self): pass
                    def wait_recv(self): pass
                return _Noop()
            i = 3 * ph + (d - 1)
            return pltpu.make_async_remote_copy(
                src_ref=src, dst_ref=dst,
                send_sem=send_sems.at[i], recv_sem=recv_sems.at[i],
                device_id=(my ^ d,), device_id_type=pl.DeviceIdType.MESH,
            )

        def start_allgather():
            ph = phase_ctr[0]
            phase_ctr[0] += 1
            rdmas = {}
            for d in (1, 2, 3):
                rdmas[d] = make_rdma(ph, d, xfull.at[my_rows, :],
                                     xfull.at[my_rows, :])
                rdmas[d].start()
            return rdmas

        def rows_out(win_b, wout_b, base, nrows):
            hh = jnp.maximum(
                jnp.dot(xfull[pl.ds(base, nrows), :], win_b,
                        preferred_element_type=jnp.float32), 0.0)
            return jnp.dot(hh.astype(jnp.bfloat16), wout_b,
                           preferred_element_type=jnp.float32)

        def stage_and_send(ph, d):
            sbuf[d - 1, :, :] = part[pl.ds((my ^ d) * B, B), :].astype(
                jnp.bfloat16)
            r = make_rdma(ph, d, sbuf.at[d - 1], rbuf.at[d - 1])
            r.start()
            return r

        layers = ((win0, wout0), (win1, wout1), (win2, wout2))

        xfull[my_rows, :] = x_ref[:, :].astype(jnp.bfloat16)
        ag = start_allgather()
        win_b = layers[0][0][:, :].astype(jnp.bfloat16)
        wout_b = layers[0][1][:, :].astype(jnp.bfloat16)
        part[my_rows, :] = rows_out(win_b, wout_b, my * B, B)

        for k in range(len(layers)):
            rs = {}
            ph = phase_ctr[0]
            phase_ctr[0] += 1
            ag[1].wait_recv()
            part[pl.ds((my ^ 1) * B, B), :] = rows_out(win_b, wout_b,
                                                       (my ^ 1) * B, B)
            rs[1] = stage_and_send(ph, 1)
            ag[3].wait_recv()
            ag[2].wait_recv()
            part[pl.ds(other_base, 2 * B), :] = rows_out(win_b, wout_b,
                                                         other_base, 2 * B)
            rs[3] = stage_and_send(ph, 3)
            rs[2] = stage_and_send(ph, 2)
            for d in (1, 3, 2):
                rs[d].wait_recv()
            red = (part[my_rows, :]
                   + rbuf[0, :, :].astype(jnp.float32)
                   + rbuf[1, :, :].astype(jnp.float32)
                   + rbuf[2, :, :].astype(jnp.float32))
            for d in (1, 2, 3):
                ag[d].wait_send()
                rs[d].wait_send()
            if k < len(layers) - 1:
                xfull[my_rows, :] = red.astype(jnp.bfloat16)
                ag = start_allgather()
                win_b = layers[k + 1][0][:, :].astype(jnp.bfloat16)
                wout_b = layers[k + 1][1][:, :].astype(jnp.bfloat16)
                part[my_rows, :] = rows_out(win_b, wout_b, my * B, B)
            else:
                out_ref[:, :] = red

    return pl.pallas_call(
        body,
        out_shape=jax.ShapeDtypeStruct((B, D), jnp.float32),
        in_specs=[pl.BlockSpec(memory_space=pltpu.VMEM)] * 7,
        out_specs=pl.BlockSpec(memory_space=pltpu.VMEM),
        scratch_shapes=[
            pltpu.VMEM((BG, D), jnp.bfloat16),
            pltpu.VMEM((BG, D), jnp.float32),
            pltpu.VMEM((3, B, D), jnp.bfloat16),
            pltpu.VMEM((3, B, D), jnp.bfloat16),
            pltpu.SemaphoreType.DMA((N_SEM,)),
            pltpu.SemaphoreType.DMA((N_SEM,)),
        ],
        compiler_params=pltpu.CompilerParams(
            collective_id=0,
            vmem_limit_bytes=100 * 1024 * 1024,
        ),
    )(x, Win0, Wout0, Win1, Wout1, Win2, Wout2)


# device time: 56966 ns/iter; 1.0589x vs baseline; 1.0477x over previous
import jax
import jax.numpy as jnp
from jax import lax
from jax.experimental import pallas as pl
from jax.experimental.pallas import tpu as pltpu

N_DEV = 4
B = 64
D = 1024
BG = N_DEV * B
N_PHASE = 6
N_SEM = 3 * N_PHASE


def kernel(x, Win0, Wout0, Win1, Wout1, Win2, Wout2):
    def body(x_ref, win0, wout0, win1, wout1, win2, wout2, out_ref,
             xfull, part, sbuf, rbuf, send_sems, recv_sems):
        my = lax.axis_index("i")

        barrier = pltpu.get_barrier_semaphore()
        for d in (1, 2, 3):
            pl.semaphore_signal(barrier, inc=1, device_id=(my ^ d,),
                                device_id_type=pl.DeviceIdType.MESH)
        pl.semaphore_wait(barrier, 3)

        phase_ctr = [0]

        def make_rdma(ph, d, src, dst):
            import os
            if os.environ.get("SKIP_COMM"):
                class _Noop:
                    def start(self): pass
                    def wait(self): pass
                    def wait_send(self): pass
                    def wait_recv(self): pass
                return _Noop()
            i = 3 * ph + (d - 1)
            return pltpu.make_async_remote_copy(
                src_ref=src, dst_ref=dst,
                send_sem=send_sems.at[i], recv_sem=recv_sems.at[i],
                device_id=(my ^ d,), device_id_type=pl.DeviceIdType.MESH,
            )

        def one_shot(srcs, dsts):
            ph = phase_ctr[0]
            phase_ctr[0] += 1
            rdmas = [make_rdma(ph, d, srcs(d), dsts(d)) for d in (1, 2, 3)]
            for r in rdmas:
                r.start()
            for r in rdmas:
                r.wait()

        def allgather():
            one_shot(lambda d: xfull.at[0], lambda d: xfull.at[d])

        def reduce_scatter():
            for d in (1, 2, 3):
                sbuf[d - 1, :, :] = part[d, :, :].astype(jnp.bfloat16)
            one_shot(lambda d: sbuf.at[d - 1], lambda d: rbuf.at[d - 1])

        xfull[0, :, :] = x_ref[:, :].astype(jnp.bfloat16)
        allgather()

        layers = ((win0, wout0), (win1, wout1), (win2, wout2))
        for k, (win, wout) in enumerate(layers):
            win_b = win[:, :].astype(jnp.bfloat16)
            wout_b = wout[:, :].astype(jnp.bfloat16)
            xf = xfull[:, :, :].reshape(BG, D)
            h = jnp.maximum(
                jnp.dot(xf, win_b, preferred_element_type=jnp.float32), 0.0)
            part[:, :, :] = jnp.dot(
                h.astype(jnp.bfloat16), wout_b,
                preferred_element_type=jnp.float32).reshape(N_DEV, B, D)
            reduce_scatter()
            red = (part[0, :, :]
                   + rbuf[0, :, :].astype(jnp.float32)
                   + rbuf[1, :, :].astype(jnp.float32)
                   + rbuf[2, :, :].astype(jnp.float32))
            if k < len(layers) - 1:
                xfull[0, :, :] = red.astype(jnp.bfloat16)
                allgather()
            else:
                out_ref[:, :] = red

    return pl.pallas_call(
        body,
        out_shape=jax.ShapeDtypeStruct((B, D), jnp.float32),
        in_specs=[pl.BlockSpec(memory_space=pltpu.VMEM)] * 7,
        out_specs=pl.BlockSpec(memory_space=pltpu.VMEM),
        scratch_shapes=[
            pltpu.VMEM((N_DEV, B, D), jnp.bfloat16),
            pltpu.VMEM((N_DEV, B, D), jnp.float32),
            pltpu.VMEM((3, B, D), jnp.bfloat16),
            pltpu.VMEM((3, B, D), jnp.bfloat16),
            pltpu.SemaphoreType.DMA((N_SEM,)),
            pltpu.SemaphoreType.DMA((N_SEM,)),
        ],
        compiler_params=pltpu.CompilerParams(
            collective_id=0,
            vmem_limit_bytes=100 * 1024 * 1024,
        ),
    )(x, Win0, Wout0, Win1, Wout1, Win2, Wout2)


# device time: 39891 ns/iter; 1.5121x vs baseline; 1.4280x over previous
import jax
import jax.numpy as jnp
from jax import lax
from jax.experimental import pallas as pl
from jax.experimental.pallas import tpu as pltpu

N_DEV = 4
B = 64
D = 1024
H = 2048
HALF = D // 2
BG = N_DEV * B
N_SLOT = 8
N_SEM = 3 * N_SLOT


def kernel(x, Win0, Wout0, Win1, Wout1, Win2, Wout2):
    def body(x_ref, win0, wout0, win1, wout1, win2, wout2, out_ref,
             xfull, xcat, part, sbuf, rbuf, winv, woutv,
             send_sems, recv_sems, wsems):
        my = lax.axis_index("i")

        barrier = pltpu.get_barrier_semaphore()
        for d in (1, 2, 3):
            pl.semaphore_signal(barrier, inc=1, device_id=(my ^ d,),
                                device_id_type=pl.DeviceIdType.MESH)
        pl.semaphore_wait(barrier, 3)

        hbm_w = ((win0, wout0), (win1, wout1), (win2, wout2))

        def start_wcopy(k):
            cin = pltpu.make_async_copy(hbm_w[k][0], winv.at[k % 2],
                                        wsems.at[2 * k])
            cout = pltpu.make_async_copy(hbm_w[k][1], woutv.at[k % 2],
                                         wsems.at[2 * k + 1])
            cin.start()
            cout.start()
            return (cin, cout)

        phase_ctr = [0]

        def make_rdma(ph, d, src, dst):
            import os
            if os.environ.get("SKIP_COMM"):
                class _Noop:
                    def start(self): pass
                    def wait(self): pass
                    def wait_send(self): pass
                    def wait_recv(self): pass
                return _Noop()
            i = 3 * (ph % N_SLOT) + (d - 1)
            return pltpu.make_async_remote_copy(
                src_ref=src, dst_ref=dst,
                send_sem=send_sems.at[i], recv_sem=recv_sems.at[i],
                device_id=(my ^ d,), device_id_type=pl.DeviceIdType.MESH,
            )

        def start_ag(c):
            ph = phase_ctr[0]
            phase_ctr[0] += 1
            rdmas = {}
            for d in (1, 2, 3):
                rdmas[d] = make_rdma(ph, d, xfull.at[c, 0], xfull.at[c, d])
                rdmas[d].start()
            return rdmas

        def start_rs(c):
            ph = phase_ctr[0]
            phase_ctr[0] += 1
            rdmas = {}
            for d in (1, 2, 3):
                sbuf[c, d - 1, :, :] = part[c, d, :, :].astype(jnp.bfloat16)
            for d in (1, 2, 3):
                rdmas[d] = make_rdma(ph, d, sbuf.at[c, d - 1],
                                     rbuf.at[c, d - 1])
                rdmas[d].start()
            return rdmas

        def reduce(c, rs):
            for d in (1, 3, 2):
                rs[d].wait_recv()
            return (part[c, 0, :, :]
                    + rbuf[c, 0, :, :].astype(jnp.float32)
                    + rbuf[c, 1, :, :].astype(jnp.float32)
                    + rbuf[c, 2, :, :].astype(jnp.float32))

        wc0 = start_wcopy(0)
        wc1 = start_wcopy(1)
        xb = x_ref[:, :].astype(jnp.bfloat16)
        xfull[0, 0, :, :] = xb[:, :HALF]
        xfull[1, 0, :, :] = xb[:, HALF:]
        ag = {0: start_ag(0), 1: start_ag(1)}
        wcopies = [wc0, wc1]

        for k in range(3):
            for c in wcopies[k]:
                c.wait()
            winb = winv.at[k % 2]
            woutb = woutv.at[k % 2]
            for d in (1, 3, 2):
                ag[0][d].wait_recv()
                ag[1][d].wait_recv()
            xcat[:, :HALF] = xfull[0, :, :, :].reshape(BG, HALF)
            xcat[:, HALF:] = xfull[1, :, :, :].reshape(BG, HALF)
            h = jnp.maximum(
                jnp.dot(xcat[:, :].astype(jnp.float32), winb[:, :],
                        preferred_element_type=jnp.float32), 0.0)
            for d in (1, 2, 3):
                ag[0][d].wait_send()
                ag[1][d].wait_send()
            part[0, :, :, :] = jnp.dot(
                h, woutb[:, :HALF],
                preferred_element_type=jnp.float32).reshape(N_DEV, B, HALF)
            rs0 = start_rs(0)
            part[1, :, :, :] = jnp.dot(
                h, woutb[:, HALF:],
                preferred_element_type=jnp.float32).reshape(N_DEV, B, HALF)
            rs1 = start_rs(1)
            if k == 0:
                wcopies.append(start_wcopy(2))
            red0 = reduce(0, rs0)
            if k < 2:
                xfull[0, 0, :, :] = red0.astype(jnp.bfloat16)
                ag = {0: start_ag(0)}
                red1 = reduce(1, rs1)
                xfull[1, 0, :, :] = red1.astype(jnp.bfloat16)
                ag[1] = start_ag(1)
            else:
                out_ref[:, :HALF] = red0
                red1 = reduce(1, rs1)
                out_ref[:, HALF:] = red1
            for d in (1, 2, 3):
                rs0[d].wait_send()
                rs1[d].wait_send()

    return pl.pallas_call(
        body,
        out_shape=jax.ShapeDtypeStruct((B, D), jnp.float32),
        in_specs=[pl.BlockSpec(memory_space=pltpu.VMEM)]
        + [pl.BlockSpec(memory_space=pltpu.MemorySpace.HBM)] * 6,
        out_specs=pl.BlockSpec(memory_space=pltpu.VMEM),
        scratch_shapes=[
            pltpu.VMEM((2, N_DEV, B, HALF), jnp.bfloat16),
            pltpu.VMEM((BG, D), jnp.bfloat16),
            pltpu.VMEM((2, N_DEV, B, HALF), jnp.float32),
            pltpu.VMEM((2, 3, B, HALF), jnp.bfloat16),
            pltpu.VMEM((2, 3, B, HALF), jnp.bfloat16),
            pltpu.VMEM((2, D, H), jnp.float32),
            pltpu.VMEM((2, H, D), jnp.float32),
            pltpu.SemaphoreType.DMA((N_SEM,)),
            pltpu.SemaphoreType.DMA((N_SEM,)),
            pltpu.SemaphoreType.DMA((6,)),
        ],
        compiler_params=pltpu.CompilerParams(
            collective_id=0,
            vmem_limit_bytes=100 * 1024 * 1024,
        ),
    )(x, Win0, Wout0, Win1, Wout1, Win2, Wout2)
